# Initial kernel scaffold; baseline (speedup 1.0000x reference)
import jax
import jax.numpy as jnp
from jax.experimental import pallas as pl


def kernel(x, o_adj, s_adj, idx, W_o1, b_o1, W_s1o, b_s1o, W_s1, b_s1, W_o1s, b_o1s, W_o2, b_o2, W_s2o, b_s2o, Wd1, bd1, Wd2, bd2):
    raise NotImplementedError("write your pallas kernel here")


# speedup vs baseline: 1.5202x; 1.5202x over previous
import functools

import jax
import jax.numpy as jnp
from jax import lax
from jax.experimental import pallas as pl
from jax.experimental.pallas import tpu as pltpu
from jax.experimental.pallas import tpu_sc as plsc

_BM = 200
_BB = 2048


def _proj_body(x_ref, w_ref, o_ref):
    o_ref[...] = jnp.dot(x_ref[...], w_ref[...],
                         preferred_element_type=jnp.float32)


def _pass1_body(oa_ref, sa_ref, u_ref, b1_ref, bt_ref, w_ref, oxw_ref, t1_ref):
    h = b1_ref.shape[1]
    p = jnp.dot(oa_ref[...], u_ref[:, 0:h], preferred_element_type=jnp.float32)
    q = jnp.dot(sa_ref[...], u_ref[:, h:3 * h],
                preferred_element_type=jnp.float32)
    o_x = jnp.maximum(p + q[:, 0:h] + b1_ref[...], 0.0)
    t1_ref[...] = q[:, h:2 * h] + bt_ref[...]
    oxw_ref[...] = jnp.dot(o_x, w_ref[...], preferred_element_type=jnp.float32)


def _pass2_body(oa_ref, u_ref, t1_ref, b1_ref, b2_ref, w_ref, sxw_ref, t2_ref):
    h = b1_ref.shape[1]
    r = jnp.dot(oa_ref[...], u_ref[...], preferred_element_type=jnp.float32)
    s_x = jnp.maximum(t1_ref[...] + r[:, 0:h] + b1_ref[...], 0.0)
    t2_ref[...] = r[:, h:2 * h] + b2_ref[...]
    sxw_ref[...] = jnp.dot(s_x, w_ref[...], preferred_element_type=jnp.float32)


def _pass3_body(sa_ref, u_ref, t2_ref, b_ref, h_ref):
    s = jnp.dot(sa_ref[...], u_ref[...], preferred_element_type=jnp.float32)
    hv = t2_ref[...] + s + b_ref[...]
    h_ref[...] = jnp.concatenate([hv, jnp.zeros_like(hv)], axis=1)


def _dec_body(f1_ref, f2_ref, w1t_ref, w1b_ref, b1_ref, w2_ref, b2_ref, o_ref):
    h = w1t_ref.shape[0]
    t = (jnp.dot(f1_ref[:, 0:h], w1t_ref[...],
                 preferred_element_type=jnp.float32)
         + jnp.dot(f2_ref[:, 0:h], w1b_ref[...],
                   preferred_element_type=jnp.float32)
         + b1_ref[...])
    o_ref[...] = jnp.dot(t, w2_ref[...],
                         preferred_element_type=jnp.float32) + b2_ref[...]


@functools.lru_cache(maxsize=None)
def _build_gather(n, h, bsz):
    info = plsc.get_sparse_core_info()
    nc, ns = info.num_cores, info.num_subcores
    nw = nc * ns
    bpw = bsz // nw
    ch = 128
    nch = bpw // ch
    mesh = plsc.VectorSubcoreMesh(core_axis_name="c", subcore_axis_name="s")

    @functools.partial(
        pl.kernel, mesh=mesh,
        out_type=[jax.ShapeDtypeStruct((bsz, 2 * h), jnp.float32),
                  jax.ShapeDtypeStruct((bsz, 2 * h), jnp.float32)],
        scratch_types=[
            pltpu.VMEM((nch, ch), jnp.int32),
            pltpu.VMEM((nch, ch), jnp.int32),
            pltpu.VMEM((ch, 2 * h), jnp.float32),
            pltpu.VMEM((ch, 2 * h), jnp.float32),
            pltpu.SemaphoreType.DMA,
            pltpu.SemaphoreType.DMA,
        ],
    )
    def gather2(t_hbm, i0_hbm, i1_hbm, o0_hbm, o1_hbm,
                i0_v, i1_v, r0_v, r1_v, s0, s1):
        wid = lax.axis_index("s") * nc + lax.axis_index("c")
        base = wid * bpw
        for k in range(nch):
            pltpu.sync_copy(i0_hbm.at[pl.ds(base + k * ch, ch)], i0_v.at[k])
            pltpu.sync_copy(i1_hbm.at[pl.ds(base + k * ch, ch)], i1_v.at[k])
        for k in range(nch):
            c0 = pltpu.async_copy(t_hbm.at[i0_v.at[k]], r0_v, s0)
            c1 = pltpu.async_copy(t_hbm.at[i1_v.at[k]], r1_v, s1)
            c0.wait()
            c1.wait()
            pltpu.sync_copy(r0_v, o0_hbm.at[pl.ds(base + k * ch, ch)])
            pltpu.sync_copy(r1_v, o1_hbm.at[pl.ds(base + k * ch, ch)])

    return gather2


def kernel(x, o_adj, s_adj, idx,
           W_o1, b_o1, W_s1o, b_s1o, W_s1, b_s1, W_o1s, b_o1s,
           W_o2, b_o2, W_s2o, b_s2o, Wd1, bd1, Wd2, bd2):
    n, _ = x.shape
    h = W_o1.shape[1]
    bsz = idx.shape[1]
    rell = Wd2.shape[1]
    g = n // _BM

    row2 = lambda i: (i, 0)
    const2 = lambda i: (0, 0)

    u0 = pl.pallas_call(
        _proj_body,
        out_shape=jax.ShapeDtypeStruct((n, 3 * h), jnp.float32),
    )(x, jnp.concatenate([W_o1, W_s1o, W_s1], axis=1))

    oxw, t1 = pl.pallas_call(
        _pass1_body,
        grid=(g,),
        in_specs=[
            pl.BlockSpec((_BM, n), row2),
            pl.BlockSpec((_BM, n), row2),
            pl.BlockSpec((n, 3 * h), const2),
            pl.BlockSpec((1, h), const2),
            pl.BlockSpec((1, h), const2),
            pl.BlockSpec((h, 2 * h), const2),
        ],
        out_specs=[pl.BlockSpec((_BM, 2 * h), row2),
                   pl.BlockSpec((_BM, h), row2)],
        out_shape=[jax.ShapeDtypeStruct((n, 2 * h), jnp.float32),
                   jax.ShapeDtypeStruct((n, h), jnp.float32)],
    )(o_adj, s_adj, u0,
      (b_o1 + b_s1o).reshape(1, h), b_s1.reshape(1, h),
      jnp.concatenate([W_o1s, W_o2], axis=1))

    sxw, t2 = pl.pallas_call(
        _pass2_body,
        grid=(g,),
        in_specs=[
            pl.BlockSpec((_BM, n), row2),
            pl.BlockSpec((n, 2 * h), const2),
            pl.BlockSpec((_BM, h), row2),
            pl.BlockSpec((1, h), const2),
            pl.BlockSpec((1, h), const2),
            pl.BlockSpec((h, h), const2),
        ],
        out_specs=[pl.BlockSpec((_BM, h), row2),
                   pl.BlockSpec((_BM, h), row2)],
        out_shape=[jax.ShapeDtypeStruct((n, h), jnp.float32),
                   jax.ShapeDtypeStruct((n, h), jnp.float32)],
    )(o_adj, oxw, t1, b_o1s.reshape(1, h), b_o2.reshape(1, h), W_s2o)

    h_nodes = pl.pallas_call(
        _pass3_body,
        grid=(g,),
        in_specs=[
            pl.BlockSpec((_BM, n), row2),
            pl.BlockSpec((n, h), const2),
            pl.BlockSpec((_BM, h), row2),
            pl.BlockSpec((1, h), const2),
        ],
        out_specs=pl.BlockSpec((_BM, 2 * h), row2),
        out_shape=jax.ShapeDtypeStruct((n, 2 * h), jnp.float32),
    )(s_adj, sxw, t2, b_s2o.reshape(1, h))

    idx32 = idx.astype(jnp.int32)
    f1, f2 = _build_gather(n, h, bsz)(h_nodes, idx32[0], idx32[1])

    gb = bsz // _BB
    o = pl.pallas_call(
        _dec_body,
        grid=(gb,),
        in_specs=[
            pl.BlockSpec((_BB, 2 * h), row2),
            pl.BlockSpec((_BB, 2 * h), row2),
            pl.BlockSpec((h, h), const2),
            pl.BlockSpec((h, h), const2),
            pl.BlockSpec((1, h), const2),
            pl.BlockSpec((h, rell), const2),
            pl.BlockSpec((1, rell), const2),
        ],
        out_specs=pl.BlockSpec((_BB, rell), row2),
        out_shape=jax.ShapeDtypeStruct((bsz, rell), jnp.float32),
    )(f1, f2, Wd1[0:h], Wd1[h:2 * h], bd1.reshape(1, h),
      Wd2, bd2.reshape(1, rell))

    return o


# speedup vs baseline: 1.5440x; 1.0156x over previous
import functools

import jax
import jax.numpy as jnp
from jax import lax
from jax.experimental import pallas as pl
from jax.experimental.pallas import tpu as pltpu
from jax.experimental.pallas import tpu_sc as plsc

_BM = 200
_BB = 2048


def _proj_body(x_ref, w_ref, o_ref):
    o_ref[...] = jnp.dot(x_ref[...], w_ref[...],
                         preferred_element_type=jnp.float32)


def _pass1_body(oa_ref, sa_ref, u_ref, b1_ref, bt_ref, w_ref, oxw_ref, t1_ref):
    h = b1_ref.shape[1]
    p = jnp.dot(oa_ref[...], u_ref[:, 0:h], preferred_element_type=jnp.float32)
    q = jnp.dot(sa_ref[...], u_ref[:, h:3 * h],
                preferred_element_type=jnp.float32)
    o_x = jnp.maximum(p + q[:, 0:h] + b1_ref[...], 0.0)
    t1_ref[...] = q[:, h:2 * h] + bt_ref[...]
    oxw_ref[...] = jnp.dot(o_x, w_ref[...], preferred_element_type=jnp.float32)


def _pass2_body(oa0_ref, oa1_ref, u_ref, t1_ref, b1_ref, b2_ref, w_ref,
                sxw_ref, t2_ref):
    h = b1_ref.shape[1]
    bm = oa0_ref.shape[0]
    r0 = jnp.dot(oa0_ref[...], u_ref[...], preferred_element_type=jnp.float32)
    r1 = jnp.dot(oa1_ref[...], u_ref[...], preferred_element_type=jnp.float32)
    r = jnp.concatenate([r0, r1], axis=0)
    s_x = jnp.maximum(t1_ref[...] + r[:, 0:h] + b1_ref[...], 0.0)
    t2_ref[...] = r[:, h:2 * h] + b2_ref[...]
    sxw_ref[...] = jnp.dot(s_x, w_ref[...], preferred_element_type=jnp.float32)


def _pass3_body(sa0_ref, sa1_ref, u_ref, t2_ref, b_ref, h_ref):
    s0 = jnp.dot(sa0_ref[...], u_ref[...], preferred_element_type=jnp.float32)
    s1 = jnp.dot(sa1_ref[...], u_ref[...], preferred_element_type=jnp.float32)
    s = jnp.concatenate([s0, s1], axis=0)
    hv = t2_ref[...] + s + b_ref[...]
    h_ref[...] = jnp.concatenate([hv, jnp.zeros_like(hv)], axis=1)


def _dec_body(f1_ref, f2_ref, w1t_ref, w1b_ref, b1_ref, w2_ref, b2_ref, o_ref):
    h = w1t_ref.shape[0]
    t = (jnp.dot(f1_ref[:, 0:h], w1t_ref[...],
                 preferred_element_type=jnp.float32)
         + jnp.dot(f2_ref[:, 0:h], w1b_ref[...],
                   preferred_element_type=jnp.float32)
         + b1_ref[...])
    o_ref[...] = jnp.dot(t, w2_ref[...],
                         preferred_element_type=jnp.float32) + b2_ref[...]


@functools.lru_cache(maxsize=None)
def _build_gather(n, h, bsz):
    info = plsc.get_sparse_core_info()
    nc, ns = info.num_cores, info.num_subcores
    nw = nc * ns
    bpw = bsz // nw
    ch = 128
    nch = bpw // ch
    mesh = plsc.VectorSubcoreMesh(core_axis_name="c", subcore_axis_name="s")

    @functools.partial(
        pl.kernel, mesh=mesh,
        out_type=[jax.ShapeDtypeStruct((bsz, 2 * h), jnp.float32),
                  jax.ShapeDtypeStruct((bsz, 2 * h), jnp.float32)],
        scratch_types=[
            pltpu.VMEM((nch, ch), jnp.int32),
            pltpu.VMEM((nch, ch), jnp.int32),
            pltpu.VMEM((ch, 2 * h), jnp.float32),
            pltpu.VMEM((ch, 2 * h), jnp.float32),
            pltpu.SemaphoreType.DMA,
            pltpu.SemaphoreType.DMA,
        ],
    )
    def gather2(t_hbm, i0_hbm, i1_hbm, o0_hbm, o1_hbm,
                i0_v, i1_v, r0_v, r1_v, s0, s1):
        wid = lax.axis_index("s") * nc + lax.axis_index("c")
        base = wid * bpw
        for k in range(nch):
            pltpu.sync_copy(i0_hbm.at[pl.ds(base + k * ch, ch)], i0_v.at[k])
            pltpu.sync_copy(i1_hbm.at[pl.ds(base + k * ch, ch)], i1_v.at[k])
        for k in range(nch):
            c0 = pltpu.async_copy(t_hbm.at[i0_v.at[k]], r0_v, s0)
            c1 = pltpu.async_copy(t_hbm.at[i1_v.at[k]], r1_v, s1)
            c0.wait()
            c1.wait()
            pltpu.sync_copy(r0_v, o0_hbm.at[pl.ds(base + k * ch, ch)])
            pltpu.sync_copy(r1_v, o1_hbm.at[pl.ds(base + k * ch, ch)])

    return gather2


def kernel(x, o_adj, s_adj, idx,
           W_o1, b_o1, W_s1o, b_s1o, W_s1, b_s1, W_o1s, b_o1s,
           W_o2, b_o2, W_s2o, b_s2o, Wd1, bd1, Wd2, bd2):
    n, _ = x.shape
    h = W_o1.shape[1]
    bsz = idx.shape[1]
    rell = Wd2.shape[1]
    g = n // _BM

    row2 = lambda i: (i, 0)
    const2 = lambda i: (0, 0)

    u0 = pl.pallas_call(
        _proj_body,
        out_shape=jax.ShapeDtypeStruct((n, 3 * h), jnp.float32),
    )(x, jnp.concatenate([W_o1, W_s1o, W_s1], axis=1))

    oxw, t1 = pl.pallas_call(
        _pass1_body,
        grid=(g,),
        in_specs=[
            pl.BlockSpec((_BM, n), row2),
            pl.BlockSpec((_BM, n), row2),
            pl.BlockSpec((n, 3 * h), const2),
            pl.BlockSpec((1, h), const2),
            pl.BlockSpec((1, h), const2),
            pl.BlockSpec((h, 2 * h), const2),
        ],
        out_specs=[pl.BlockSpec((_BM, 2 * h), row2),
                   pl.BlockSpec((_BM, h), row2)],
        out_shape=[jax.ShapeDtypeStruct((n, 2 * h), jnp.float32),
                   jax.ShapeDtypeStruct((n, h), jnp.float32)],
    )(o_adj, s_adj, u0,
      (b_o1 + b_s1o).reshape(1, h), b_s1.reshape(1, h),
      jnp.concatenate([W_o1s, W_o2], axis=1))

    even = lambda i: (2 * i, 0)
    odd = lambda i: (2 * i + 1, 0)
    sxw, t2 = pl.pallas_call(
        _pass2_body,
        grid=(g // 2,),
        in_specs=[
            pl.BlockSpec((_BM, n), even),
            pl.BlockSpec((_BM, n), odd),
            pl.BlockSpec((n, 2 * h), const2),
            pl.BlockSpec((2 * _BM, h), row2),
            pl.BlockSpec((1, h), const2),
            pl.BlockSpec((1, h), const2),
            pl.BlockSpec((h, h), const2),
        ],
        out_specs=[pl.BlockSpec((2 * _BM, h), row2),
                   pl.BlockSpec((2 * _BM, h), row2)],
        out_shape=[jax.ShapeDtypeStruct((n, h), jnp.float32),
                   jax.ShapeDtypeStruct((n, h), jnp.float32)],
    )(o_adj, o_adj, oxw, t1, b_o1s.reshape(1, h), b_o2.reshape(1, h), W_s2o)

    h_nodes = pl.pallas_call(
        _pass3_body,
        grid=(g // 2,),
        in_specs=[
            pl.BlockSpec((_BM, n), even),
            pl.BlockSpec((_BM, n), odd),
            pl.BlockSpec((n, h), const2),
            pl.BlockSpec((2 * _BM, h), row2),
            pl.BlockSpec((1, h), const2),
        ],
        out_specs=pl.BlockSpec((2 * _BM, 2 * h), row2),
        out_shape=jax.ShapeDtypeStruct((n, 2 * h), jnp.float32),
    )(s_adj, s_adj, sxw, t2, b_s2o.reshape(1, h))

    idx32 = idx.astype(jnp.int32)
    f1, f2 = _build_gather(n, h, bsz)(h_nodes, idx32[0], idx32[1])

    gb = bsz // _BB
    o = pl.pallas_call(
        _dec_body,
        grid=(gb,),
        in_specs=[
            pl.BlockSpec((_BB, 2 * h), row2),
            pl.BlockSpec((_BB, 2 * h), row2),
            pl.BlockSpec((h, h), const2),
            pl.BlockSpec((h, h), const2),
            pl.BlockSpec((1, h), const2),
            pl.BlockSpec((h, rell), const2),
            pl.BlockSpec((1, rell), const2),
        ],
        out_specs=pl.BlockSpec((_BB, rell), row2),
        out_shape=jax.ShapeDtypeStruct((bsz, rell), jnp.float32),
    )(f1, f2, Wd1[0:h], Wd1[h:2 * h], bd1.reshape(1, h),
      Wd2, bd2.reshape(1, rell))

    return o


# speedup vs baseline: 1.5710x; 1.0175x over previous
import functools

import jax
import jax.numpy as jnp
from jax import lax
from jax.experimental import pallas as pl
from jax.experimental.pallas import tpu as pltpu
from jax.experimental.pallas import tpu_sc as plsc

_BM = 200
_BB = 2048


def _pass1_body(x_ref, wc_ref, oa_ref, sa_ref, b1_ref, bt_ref, w_ref,
                oxw_ref, t1_ref, u_scr):
    h = b1_ref.shape[1]

    @pl.when(pl.program_id(0) == 0)
    def _proj():
        u_scr[...] = jnp.dot(x_ref[...], wc_ref[...],
                             preferred_element_type=jnp.float32)

    p = jnp.dot(oa_ref[...], u_scr[:, 0:h], preferred_element_type=jnp.float32)
    q = jnp.dot(sa_ref[...], u_scr[:, h:3 * h],
                preferred_element_type=jnp.float32)
    o_x = jnp.maximum(p + q[:, 0:h] + b1_ref[...], 0.0)
    t1_ref[...] = q[:, h:2 * h] + bt_ref[...]
    oxw_ref[...] = jnp.dot(o_x, w_ref[...], preferred_element_type=jnp.float32)


def _pass23_body(oa_ref, sa_ref, u_ref, t1_ref, b1_ref, b2_ref, b3_ref,
                 w_ref, h_ref, sxw_scr, t2_scr, g):
    h = b1_ref.shape[1]
    bm = oa_ref.shape[0]
    i = pl.program_id(0)

    @pl.when(i < g)
    def _phase_a():
        r = jnp.dot(oa_ref[...], u_ref[...], preferred_element_type=jnp.float32)
        s_x = jnp.maximum(t1_ref[...] + r[:, 0:h] + b1_ref[...], 0.0)
        t2_scr[pl.ds(i * bm, bm), :] = r[:, h:2 * h] + b2_ref[...]
        sxw_scr[pl.ds(i * bm, bm), :] = jnp.dot(
            s_x, w_ref[...], preferred_element_type=jnp.float32)

    @pl.when(i >= g)
    def _phase_b():
        j = i - g
        s = jnp.dot(sa_ref[...], sxw_scr[...],
                    preferred_element_type=jnp.float32)
        hv = t2_scr[pl.ds(j * bm, bm), :] + s + b3_ref[...]
        h_ref[...] = jnp.concatenate([hv, jnp.zeros_like(hv)], axis=1)


def _dec_body(f1_ref, f2_ref, w1t_ref, w1b_ref, b1_ref, w2_ref, b2_ref, o_ref):
    h = w1t_ref.shape[0]
    t = (jnp.dot(f1_ref[:, 0:h], w1t_ref[...],
                 preferred_element_type=jnp.float32)
         + jnp.dot(f2_ref[:, 0:h], w1b_ref[...],
                   preferred_element_type=jnp.float32)
         + b1_ref[...])
    o_ref[...] = jnp.dot(t, w2_ref[...],
                         preferred_element_type=jnp.float32) + b2_ref[...]


@functools.lru_cache(maxsize=None)
def _build_gather(n, h, bsz):
    info = plsc.get_sparse_core_info()
    nc, ns = info.num_cores, info.num_subcores
    nw = nc * ns
    bpw = bsz // nw
    ch = 128
    nch = bpw // ch
    mesh = plsc.VectorSubcoreMesh(core_axis_name="c", subcore_axis_name="s")

    @functools.partial(
        pl.kernel, mesh=mesh,
        out_type=[jax.ShapeDtypeStruct((bsz, 2 * h), jnp.float32),
                  jax.ShapeDtypeStruct((bsz, 2 * h), jnp.float32)],
        scratch_types=[
            pltpu.VMEM((nch, ch), jnp.int32),
            pltpu.VMEM((nch, ch), jnp.int32),
            pltpu.VMEM((ch, 2 * h), jnp.float32),
            pltpu.VMEM((ch, 2 * h), jnp.float32),
            pltpu.VMEM((ch, 2 * h), jnp.float32),
            pltpu.VMEM((ch, 2 * h), jnp.float32),
            pltpu.SemaphoreType.DMA,
            pltpu.SemaphoreType.DMA,
            pltpu.SemaphoreType.DMA,
            pltpu.SemaphoreType.DMA,
            pltpu.SemaphoreType.DMA,
        ],
    )
    def gather2(t_hbm, i0_hbm, i1_hbm, o0_hbm, o1_hbm,
                i0_v, i1_v, r0a_v, r1a_v, r0b_v, r1b_v,
                s0a, s1a, s0b, s1b, si):
        wid = lax.axis_index("s") * nc + lax.axis_index("c")
        base = wid * bpw
        rbufs = [(r0a_v, r1a_v), (r0b_v, r1b_v)]
        sems = [(s0a, s1a), (s0b, s1b)]

        idx_copies = []
        for k in range(nch):
            idx_copies.append(pltpu.async_copy(
                i0_hbm.at[pl.ds(base + k * ch, ch)], i0_v.at[k], si))
            idx_copies.append(pltpu.async_copy(
                i1_hbm.at[pl.ds(base + k * ch, ch)], i1_v.at[k], si))
        for c in idx_copies:
            c.wait()

        def start(k):
            r0, r1 = rbufs[k % 2]
            sa, sb = sems[k % 2]
            return (pltpu.async_copy(t_hbm.at[i0_v.at[k]], r0, sa),
                    pltpu.async_copy(t_hbm.at[i1_v.at[k]], r1, sb))

        inflight = start(0)
        for k in range(nch):
            c0, c1 = inflight
            c0.wait()
            c1.wait()
            if k + 1 < nch:
                inflight = start(k + 1)
            r0, r1 = rbufs[k % 2]
            pltpu.sync_copy(r0, o0_hbm.at[pl.ds(base + k * ch, ch)])
            pltpu.sync_copy(r1, o1_hbm.at[pl.ds(base + k * ch, ch)])

    return gather2


def kernel(x, o_adj, s_adj, idx,
           W_o1, b_o1, W_s1o, b_s1o, W_s1, b_s1, W_o1s, b_o1s,
           W_o2, b_o2, W_s2o, b_s2o, Wd1, bd1, Wd2, bd2):
    n, _ = x.shape
    h = W_o1.shape[1]
    bsz = idx.shape[1]
    rell = Wd2.shape[1]
    g = n // _BM

    row2 = lambda i: (i, 0)
    const2 = lambda i: (0, 0)

    oxw, t1 = pl.pallas_call(
        _pass1_body,
        grid=(g,),
        in_specs=[
            pl.BlockSpec((n, x.shape[1]), const2),
            pl.BlockSpec((x.shape[1], 3 * h), const2),
            pl.BlockSpec((_BM, n), row2),
            pl.BlockSpec((_BM, n), row2),
            pl.BlockSpec((1, h), const2),
            pl.BlockSpec((1, h), const2),
            pl.BlockSpec((h, 2 * h), const2),
        ],
        out_specs=[pl.BlockSpec((_BM, 2 * h), row2),
                   pl.BlockSpec((_BM, h), row2)],
        out_shape=[jax.ShapeDtypeStruct((n, 2 * h), jnp.float32),
                   jax.ShapeDtypeStruct((n, h), jnp.float32)],
        scratch_shapes=[pltpu.VMEM((n, 3 * h), jnp.float32)],
    )(x, jnp.concatenate([W_o1, W_s1o, W_s1], axis=1), o_adj, s_adj,
      (b_o1 + b_s1o).reshape(1, h), b_s1.reshape(1, h),
      jnp.concatenate([W_o1s, W_o2], axis=1))

    h_nodes = pl.pallas_call(
        functools.partial(_pass23_body, g=g),
        grid=(2 * g,),
        in_specs=[
            pl.BlockSpec((_BM, n), lambda i: (jnp.minimum(i, g - 1), 0)),
            pl.BlockSpec((_BM, n), lambda i: (jnp.maximum(i - g, 0), 0)),
            pl.BlockSpec((n, 2 * h), const2),
            pl.BlockSpec((_BM, h), lambda i: (jnp.minimum(i, g - 1), 0)),
            pl.BlockSpec((1, h), const2),
            pl.BlockSpec((1, h), const2),
            pl.BlockSpec((1, h), const2),
            pl.BlockSpec((h, h), const2),
        ],
        out_specs=pl.BlockSpec((_BM, 2 * h), lambda i: (jnp.maximum(i - g, 0), 0)),
        out_shape=jax.ShapeDtypeStruct((n, 2 * h), jnp.float32),
        scratch_shapes=[pltpu.VMEM((n, h), jnp.float32),
                        pltpu.VMEM((n, h), jnp.float32)],
    )(o_adj, s_adj, oxw, t1,
      b_o1s.reshape(1, h), b_o2.reshape(1, h), b_s2o.reshape(1, h), W_s2o)

    idx32 = idx.astype(jnp.int32)
    f1, f2 = _build_gather(n, h, bsz)(h_nodes, idx32[0], idx32[1])

    gb = bsz // _BB
    o = pl.pallas_call(
        _dec_body,
        grid=(gb,),
        in_specs=[
            pl.BlockSpec((_BB, 2 * h), row2),
            pl.BlockSpec((_BB, 2 * h), row2),
            pl.BlockSpec((h, h), const2),
            pl.BlockSpec((h, h), const2),
            pl.BlockSpec((1, h), const2),
            pl.BlockSpec((h, rell), const2),
            pl.BlockSpec((1, rell), const2),
        ],
        out_specs=pl.BlockSpec((_BB, rell), row2),
        out_shape=jax.ShapeDtypeStruct((bsz, rell), jnp.float32),
    )(f1, f2, Wd1[0:h], Wd1[h:2 * h], bd1.reshape(1, h),
      Wd2, bd2.reshape(1, rell))

    return o
